# baseline (device time: 41262 ns/iter reference)
import jax
import jax.numpy as jnp
from jax import lax
from jax.experimental import pallas as pl
from jax.experimental.pallas import tpu as pltpu

Z = 4
C = 4
N_DEV = Z * C


def kernel(A, B):
    m, _ = A.shape
    _, n = B.shape
    qch = m // C
    hq = qch // 2
    pch = qch // Z

    def body(a_ref, b_ref, out_ref, acc, srs_t1, srs_t2, zrecv,
             send_sems, srs1_r, srs2_r, zrs_r, zag_r, sag1_r, sag2_r):
        my = lax.axis_index("i")
        z = my // C
        c = my % C
        s_up = z * C + (c + 1) % C
        s_dn = z * C + (c - 1) % C
        MESH = pl.DeviceIdType.MESH

        col = [((z + k) % Z) * C + c for k in range(1, Z)]
        barrier_sem = pltpu.get_barrier_semaphore()
        for nbr in [s_dn, s_up] + col:
            pl.semaphore_signal(
                barrier_sem, inc=1, device_id=(nbr,), device_id_type=MESH
            )
        pl.semaphore_wait(barrier_sem, 5)

        def quarter(q, half=None):
            q = q % C
            if half is None:
                return pl.ds(q * qch, qch)
            return pl.ds(q * qch + half * hq, hq)

        def piece(q, j):
            return pl.ds((q % C) * qch + (j % Z) * pch, pch)

        def poff(j):
            return pl.ds((j % Z) * pch, pch)

        def mm(rows):
            acc[rows, :] = jnp.dot(
                a_ref[rows, :], b_ref[:, :],
                preferred_element_type=jnp.float32,
            ).astype(jnp.bfloat16)

        mm(quarter(c + 2, 1))
        up1 = pltpu.make_async_remote_copy(
            src_ref=acc.at[quarter(c + 2, 1), :],
            dst_ref=srs_t1.at[0],
            send_sem=send_sems.at[0],
            recv_sem=srs1_r.at[0],
            device_id=(s_up,),
            device_id_type=MESH,
        )
        up1.start()
        mm(quarter(c + 2, 0))
        dn1 = pltpu.make_async_remote_copy(
            src_ref=acc.at[quarter(c + 2, 0), :],
            dst_ref=srs_t1.at[1],
            send_sem=send_sems.at[1],
            recv_sem=srs1_r.at[1],
            device_id=(s_dn,),
            device_id_type=MESH,
        )
        dn1.start()
        mm(quarter(c + 1))
        mm(quarter(c - 1))
        up1.wait()
        dn1.wait()
        acc[quarter(c + 1, 1), :] = acc[quarter(c + 1, 1), :] + srs_t1[0]
        acc[quarter(c - 1, 0), :] = acc[quarter(c - 1, 0), :] + srs_t1[1]

        t2 = []
        for i in range(Z):
            up2 = pltpu.make_async_remote_copy(
                src_ref=acc.at[piece(c + 1, z + i), :],
                dst_ref=srs_t2.at[0, poff(z + i)],
                send_sem=send_sems.at[2 + i],
                recv_sem=srs2_r.at[i],
                device_id=(s_up,),
                device_id_type=MESH,
            )
            dn2 = pltpu.make_async_remote_copy(
                src_ref=acc.at[piece(c - 1, z + i), :],
                dst_ref=srs_t2.at[1, poff(z + i)],
                send_sem=send_sems.at[6 + i],
                recv_sem=srs2_r.at[4 + i],
                device_id=(s_dn,),
                device_id_type=MESH,
            )
            up2.start()
            dn2.start()
            t2.append((up2, dn2))
        mm(quarter(c))

        zsends = []
        for i in range(Z):
            up2, dn2 = t2[i]
            up2.wait()
            dn2.wait()
            rows = piece(c, z + i)
            acc[rows, :] = (
                acc[rows, :] + srs_t2[0, poff(z + i)] + srs_t2[1, poff(z + i)]
            )
            if i > 0:
                rdma = pltpu.make_async_remote_copy(
                    src_ref=acc.at[rows, :],
                    dst_ref=zrecv.at[i - 1],
                    send_sem=send_sems.at[10 + i],
                    recv_sem=zrs_r.at[i - 1],
                    device_id=(col[i - 1],),
                    device_id_type=MESH,
                )
                rdma.start()
                zsends.append(rdma)

        for rdma in zsends:
            rdma.wait()
        acc[piece(c, z), :] = jnp.maximum(
            acc[piece(c, z), :] + zrecv[0] + zrecv[1] + zrecv[2],
            jnp.bfloat16(0),
        )

        zag = []
        for k in range(1, Z):
            rdma = pltpu.make_async_remote_copy(
                src_ref=acc.at[piece(c, z), :],
                dst_ref=acc.at[piece(c, z), :],
                send_sem=send_sems.at[1 + k],
                recv_sem=zag_r.at[k - 1],
                device_id=(col[k - 1],),
                device_id_type=MESH,
            )
            rdma.start()
            zag.append(rdma)

        sag1 = []
        for i in range(Z):
            if i > 0:
                zag[i - 1].wait()
            rows = piece(c, z - i)
            ag_up = pltpu.make_async_remote_copy(
                src_ref=acc.at[rows, :],
                dst_ref=acc.at[rows, :],
                send_sem=send_sems.at[5 + i],
                recv_sem=sag1_r.at[i],
                device_id=(s_up,),
                device_id_type=MESH,
            )
            ag_dn = pltpu.make_async_remote_copy(
                src_ref=acc.at[rows, :],
                dst_ref=acc.at[rows, :],
                send_sem=send_sems.at[9 + i],
                recv_sem=sag1_r.at[4 + i],
                device_id=(s_dn,),
                device_id_type=MESH,
            )
            ag_up.start()
            ag_dn.start()
            sag1.append((ag_up, ag_dn))

        rows = quarter(c)
        out_ref[rows, :] = acc[rows, :].astype(jnp.float32)

        def fwd(rows, send_i, recv_i, dev):
            rdma = pltpu.make_async_remote_copy(
                src_ref=acc.at[rows, :],
                dst_ref=acc.at[rows, :],
                send_sem=send_sems.at[send_i],
                recv_sem=sag2_r.at[recv_i],
                device_id=(dev,),
                device_id_type=MESH,
            )
            rdma.start()
            return rdma

        sag1[0][1].wait_recv()
        f_dn0 = fwd(piece(c + 1, z), 2, 2, s_dn)
        sag1[1][0].wait_recv()
        f_up0 = fwd(piece(c - 1, z - 1), 3, 0, s_up)
        sag1[2][0].wait_recv()
        f_up1 = fwd(piece(c - 1, z - 2), 4, 1, s_up)
        sag1[3][1].wait_recv()
        f_dn1 = fwd(piece(c + 1, z + 1), 13, 3, s_dn)

        sag1[0][0].wait()
        sag1[0][1].wait_send()
        sag1[1][0].wait_send()
        sag1[1][1].wait()
        sag1[2][0].wait_send()
        sag1[2][1].wait()
        sag1[3][0].wait()
        sag1[3][1].wait_send()

        for q in (c + 1, c - 1):
            rows = quarter(q)
            out_ref[rows, :] = acc[rows, :].astype(jnp.float32)
        for rdma in (f_up0, f_up1, f_dn0, f_dn1):
            rdma.wait()
        rows = quarter(c + 2)
        out_ref[rows, :] = acc[rows, :].astype(jnp.float32)

    return pl.pallas_call(
        body,
        out_shape=jax.ShapeDtypeStruct((m, n), jnp.float32),
        in_specs=[
            pl.BlockSpec(memory_space=pltpu.VMEM),
            pl.BlockSpec(memory_space=pltpu.VMEM),
        ],
        out_specs=pl.BlockSpec(memory_space=pltpu.VMEM),
        scratch_shapes=[
            pltpu.VMEM((m, n), jnp.bfloat16),
            pltpu.VMEM((2, hq, n), jnp.bfloat16),
            pltpu.VMEM((2, qch, n), jnp.bfloat16),
            pltpu.VMEM((Z - 1, pch, n), jnp.bfloat16),
            pltpu.SemaphoreType.DMA((14,)),
            pltpu.SemaphoreType.DMA((2,)),
            pltpu.SemaphoreType.DMA((8,)),
            pltpu.SemaphoreType.DMA((Z - 1,)),
            pltpu.SemaphoreType.DMA((Z - 1,)),
            pltpu.SemaphoreType.DMA((8,)),
            pltpu.SemaphoreType.DMA((2,)),
        ],
        compiler_params=pltpu.CompilerParams(collective_id=0),
    )(A, B)


# device time: 39370 ns/iter; 1.0481x vs baseline; 1.0481x over previous
import jax
import jax.numpy as jnp
from jax import lax
from jax.experimental import pallas as pl
from jax.experimental.pallas import tpu as pltpu

Z = 4
C = 4
N_DEV = Z * C


def kernel(A, B):
    m, _ = A.shape
    _, n = B.shape
    qch = m // C
    hq = qch // 2
    pch = qch // Z

    def body(a_ref, b_ref, out_ref, acc, srs_t1, srs_t2, zrecv,
             send_sems, srs1_r, srs2_r, zrs_r, zag_r, sag1_r, sag2_r):
        my = lax.axis_index("i")
        z = my // C
        c = my % C
        s_up = z * C + (c + 1) % C
        s_dn = z * C + (c - 1) % C
        MESH = pl.DeviceIdType.MESH

        col = [((z + k) % Z) * C + c for k in range(1, Z)]
        barrier_sem = pltpu.get_barrier_semaphore()
        for nbr in [s_dn, s_up] + col:
            pl.semaphore_signal(
                barrier_sem, inc=1, device_id=(nbr,), device_id_type=MESH
            )
        pl.semaphore_wait(barrier_sem, 5)

        def quarter(q, half=None):
            q = q % C
            if half is None:
                return pl.ds(q * qch, qch)
            return pl.ds(q * qch + half * hq, hq)

        def piece(q, j):
            return pl.ds((q % C) * qch + (j % Z) * pch, pch)

        def poff(j):
            return pl.ds((j % Z) * pch, pch)

        def mm(rows):
            acc[rows, :] = jnp.dot(
                a_ref[rows, :], b_ref[:, :],
                preferred_element_type=jnp.float32,
            ).astype(jnp.bfloat16)

        mm(quarter(c + 2, 1))
        up1 = pltpu.make_async_remote_copy(
            src_ref=acc.at[quarter(c + 2, 1), :],
            dst_ref=srs_t1.at[0],
            send_sem=send_sems.at[0],
            recv_sem=srs1_r.at[0],
            device_id=(s_up,),
            device_id_type=MESH,
        )
        up1.start()
        mm(quarter(c + 2, 0))
        dn1 = pltpu.make_async_remote_copy(
            src_ref=acc.at[quarter(c + 2, 0), :],
            dst_ref=srs_t1.at[1],
            send_sem=send_sems.at[1],
            recv_sem=srs1_r.at[1],
            device_id=(s_dn,),
            device_id_type=MESH,
        )
        dn1.start()
        mm(quarter(c + 1))
        mm(quarter(c - 1))
        up1.wait()
        dn1.wait()
        acc[quarter(c + 1, 1), :] = acc[quarter(c + 1, 1), :] + srs_t1[0]
        acc[quarter(c - 1, 0), :] = acc[quarter(c - 1, 0), :] + srs_t1[1]

        RSPERM = (2, 1, 3, 0)
        t2 = []
        for i, p in enumerate(RSPERM):
            up2 = pltpu.make_async_remote_copy(
                src_ref=acc.at[piece(c + 1, z + p), :],
                dst_ref=srs_t2.at[0, poff(z + p)],
                send_sem=send_sems.at[2 + i],
                recv_sem=srs2_r.at[i],
                device_id=(s_up,),
                device_id_type=MESH,
            )
            dn2 = pltpu.make_async_remote_copy(
                src_ref=acc.at[piece(c - 1, z + p), :],
                dst_ref=srs_t2.at[1, poff(z + p)],
                send_sem=send_sems.at[6 + i],
                recv_sem=srs2_r.at[4 + i],
                device_id=(s_dn,),
                device_id_type=MESH,
            )
            up2.start()
            dn2.start()
            t2.append((up2, dn2))
        mm(quarter(c))

        zsends = []
        for i, p in enumerate(RSPERM):
            up2, dn2 = t2[i]
            up2.wait()
            dn2.wait()
            rows = piece(c, z + p)
            acc[rows, :] = (
                acc[rows, :] + srs_t2[0, poff(z + p)] + srs_t2[1, poff(z + p)]
            )
            if p > 0:
                rdma = pltpu.make_async_remote_copy(
                    src_ref=acc.at[rows, :],
                    dst_ref=zrecv.at[p - 1],
                    send_sem=send_sems.at[10 + p],
                    recv_sem=zrs_r.at[p - 1],
                    device_id=(col[p - 1],),
                    device_id_type=MESH,
                )
                rdma.start()
                zsends.append(rdma)

        for rdma in zsends:
            rdma.wait()
        acc[piece(c, z), :] = jnp.maximum(
            acc[piece(c, z), :] + zrecv[0] + zrecv[1] + zrecv[2],
            jnp.bfloat16(0),
        )

        zag = []
        for k in range(1, Z):
            rdma = pltpu.make_async_remote_copy(
                src_ref=acc.at[piece(c, z), :],
                dst_ref=acc.at[piece(c, z), :],
                send_sem=send_sems.at[1 + k],
                recv_sem=zag_r.at[k - 1],
                device_id=(col[k - 1],),
                device_id_type=MESH,
            )
            rdma.start()
            zag.append(rdma)

        AGPERM = (0, 1, 3, 2)
        sag1 = []
        for i, p in enumerate(AGPERM):
            if p > 0:
                zag[p - 1].wait()
            rows = piece(c, z - p)
            ag_up = pltpu.make_async_remote_copy(
                src_ref=acc.at[rows, :],
                dst_ref=acc.at[rows, :],
                send_sem=send_sems.at[5 + i],
                recv_sem=sag1_r.at[i],
                device_id=(s_up,),
                device_id_type=MESH,
            )
            ag_dn = pltpu.make_async_remote_copy(
                src_ref=acc.at[rows, :],
                dst_ref=acc.at[rows, :],
                send_sem=send_sems.at[9 + i],
                recv_sem=sag1_r.at[4 + i],
                device_id=(s_dn,),
                device_id_type=MESH,
            )
            ag_up.start()
            ag_dn.start()
            sag1.append((ag_up, ag_dn))

        rows = quarter(c)
        out_ref[rows, :] = acc[rows, :].astype(jnp.float32)

        def fwd(rows, send_i, recv_i, dev):
            rdma = pltpu.make_async_remote_copy(
                src_ref=acc.at[rows, :],
                dst_ref=acc.at[rows, :],
                send_sem=send_sems.at[send_i],
                recv_sem=sag2_r.at[recv_i],
                device_id=(dev,),
                device_id_type=MESH,
            )
            rdma.start()
            return rdma

        sag1[0][1].wait_recv()
        f_dn0 = fwd(piece(c + 1, z), 2, 2, s_dn)
        sag1[1][0].wait_recv()
        f_up0 = fwd(piece(c - 1, z - 1), 3, 0, s_up)
        sag1[2][1].wait_recv()
        f_dn1 = fwd(piece(c + 1, z + 1), 4, 3, s_dn)
        sag1[3][0].wait_recv()
        f_up1 = fwd(piece(c - 1, z - 2), 13, 1, s_up)

        sag1[0][0].wait()
        sag1[0][1].wait_send()
        sag1[1][0].wait_send()
        sag1[1][1].wait()
        sag1[2][0].wait()
        sag1[2][1].wait_send()
        sag1[3][0].wait_send()
        sag1[3][1].wait()

        for q in (c + 1, c - 1):
            rows = quarter(q)
            out_ref[rows, :] = acc[rows, :].astype(jnp.float32)
        for rdma in (f_up0, f_up1, f_dn0, f_dn1):
            rdma.wait()
        rows = quarter(c + 2)
        out_ref[rows, :] = acc[rows, :].astype(jnp.float32)

    return pl.pallas_call(
        body,
        out_shape=jax.ShapeDtypeStruct((m, n), jnp.float32),
        in_specs=[
            pl.BlockSpec(memory_space=pltpu.VMEM),
            pl.BlockSpec(memory_space=pltpu.VMEM),
        ],
        out_specs=pl.BlockSpec(memory_space=pltpu.VMEM),
        scratch_shapes=[
            pltpu.VMEM((m, n), jnp.bfloat16),
            pltpu.VMEM((2, hq, n), jnp.bfloat16),
            pltpu.VMEM((2, qch, n), jnp.bfloat16),
            pltpu.VMEM((Z - 1, pch, n), jnp.bfloat16),
            pltpu.SemaphoreType.DMA((14,)),
            pltpu.SemaphoreType.DMA((2,)),
            pltpu.SemaphoreType.DMA((8,)),
            pltpu.SemaphoreType.DMA((Z - 1,)),
            pltpu.SemaphoreType.DMA((Z - 1,)),
            pltpu.SemaphoreType.DMA((8,)),
            pltpu.SemaphoreType.DMA((2,)),
        ],
        compiler_params=pltpu.CompilerParams(collective_id=0),
    )(A, B)
